# baseline (speedup 1.0000x reference)
import jax
import jax.numpy as jnp
from jax import lax
from jax.experimental import pallas as pl
from jax.experimental.pallas import tpu as pltpu
from jax.experimental.pallas import tpu_sc as plsc

_NUM_FEATURES = 100
_EMBED_DIM = 32
_BATCH = 16384
_ROW = _NUM_FEATURES * _EMBED_DIM
_LANES = 16
_VECS = _ROW // _LANES

_NC = 2
_NS = 16
_NW = _NC * _NS
_C = 8
_U = 10

_SC_ROWS = 4096
_RPW = _SC_ROWS // _NW
_NCH = _RPW // _C
_NP = _NCH // 2

_TC_BLOCK = 256


def _sc_body(x_hbm, tab_hbm, out_hbm, tab_v, bi, bo, sin0, sin1, so0, so1):
    sins = (sin0, sin1)
    souts = (so0, so1)
    wid = lax.axis_index("s") * _NC + lax.axis_index("c")
    base = wid * _RPW
    pltpu.sync_copy(tab_hbm, tab_v)

    def start_in(g, b):
        r0 = base + g * _C
        pltpu.async_copy(x_hbm.at[pl.ds(r0, _C)], bi.at[b], sins[b])

    def wait_in(b):
        pltpu.make_async_copy(x_hbm.at[pl.ds(0, _C)], bi.at[b], sins[b]).wait()

    def start_out(g, b):
        r0 = base + g * _C
        pltpu.async_copy(bo.at[b], out_hbm.at[pl.ds(r0, _C)], souts[b])

    def wait_out(b):
        pltpu.make_async_copy(bo.at[b], out_hbm.at[pl.ds(0, _C)], souts[b]).wait()

    def compute(b):
        for jb in range(_VECS // _U):
            ts = [tab_v[pl.ds((jb * _U + u) * _LANES, _LANES)] for u in range(_U)]

            def rbody(r, c2):
                for u in range(_U):
                    off = (jb * _U + u) * _LANES
                    bo[b, r, pl.ds(off, _LANES)] = (
                        bi[b, r, pl.ds(off, _LANES)] + ts[u]
                    )
                return c2

            lax.fori_loop(0, _C, rbody, 0, unroll=2)

    start_in(0, 0)
    start_in(1, 1)
    for b in range(2):
        wait_in(b)
        compute(b)
        start_out(b, b)
        start_in(b + 2, b)

    def pair(p, carry):
        for b in range(2):
            g = p * 2 + b
            wait_in(b)
            wait_out(b)
            compute(b)
            start_out(g, b)
            nxt = jnp.minimum(g + 2, _NCH - 1)
            start_in(nxt, b)
        return carry

    lax.fori_loop(1, _NP, pair, 0)

    wait_in(0)
    wait_in(1)
    wait_out(0)
    wait_out(1)


def _sc_add(x, tab):
    mesh = plsc.VectorSubcoreMesh(core_axis_name="c", subcore_axis_name="s")
    return pl.kernel(
        _sc_body,
        out_type=jax.ShapeDtypeStruct((_SC_ROWS, _ROW), jnp.float32),
        mesh=mesh,
        scratch_types=[
            pltpu.VMEM((_ROW,), jnp.float32),
            pltpu.VMEM((2, _C, _ROW), jnp.float32),
            pltpu.VMEM((2, _C, _ROW), jnp.float32),
            pltpu.SemaphoreType.DMA,
            pltpu.SemaphoreType.DMA,
            pltpu.SemaphoreType.DMA,
            pltpu.SemaphoreType.DMA,
        ],
    )(x, tab)


def _tc_body(x_ref, t_ref, o_ref):
    o_ref[...] = x_ref[...] + t_ref[...]


def _tc_add(x, tab2):
    n = x.shape[0]
    return pl.pallas_call(
        _tc_body,
        grid=(n // _TC_BLOCK,),
        in_specs=[
            pl.BlockSpec((_TC_BLOCK, _ROW), lambda i: (i, 0)),
            pl.BlockSpec((1, _ROW), lambda i: (0, 0)),
        ],
        out_specs=pl.BlockSpec((_TC_BLOCK, _ROW), lambda i: (i, 0)),
        out_shape=jax.ShapeDtypeStruct((n, _ROW), jnp.float32),
    )(x, tab2)


def kernel(inputs, column_table):
    x = inputs.reshape(_BATCH, _ROW)
    tab = column_table.reshape(_ROW)
    sc_out = _sc_add(x[:_SC_ROWS], tab)
    tc_out = _tc_add(x[_SC_ROWS:], tab.reshape(1, _ROW))
    out = jnp.concatenate([sc_out, tc_out], axis=0)
    return out.reshape(_BATCH, _NUM_FEATURES, _EMBED_DIM)


# speedup vs baseline: 1.2846x; 1.2846x over previous
import jax
import jax.numpy as jnp
from jax import lax
from jax.experimental import pallas as pl
from jax.experimental.pallas import tpu as pltpu
from jax.experimental.pallas import tpu_sc as plsc

_NUM_FEATURES = 100
_EMBED_DIM = 32
_BATCH = 16384
_ROW = _NUM_FEATURES * _EMBED_DIM
_LANES = 16
_VECS = _ROW // _LANES

_NC = 2
_NS = 16
_NW = _NC * _NS
_C = 8
_U = 10

_SC_ROWS = 16384
_RPW = _SC_ROWS // _NW
_NCH = _RPW // _C
_NP = _NCH // 2

_TC_BLOCK = 256


def _sc_body(x_hbm, tab_hbm, out_hbm, tab_v, bi, bo, sin0, sin1, so0, so1):
    sins = (sin0, sin1)
    souts = (so0, so1)
    wid = lax.axis_index("s") * _NC + lax.axis_index("c")
    base = wid * _RPW
    pltpu.sync_copy(tab_hbm, tab_v)

    def start_in(g, b):
        r0 = base + g * _C
        pltpu.async_copy(x_hbm.at[pl.ds(r0, _C)], bi.at[b], sins[b])

    def wait_in(b):
        pltpu.make_async_copy(x_hbm.at[pl.ds(0, _C)], bi.at[b], sins[b]).wait()

    def start_out(g, b):
        r0 = base + g * _C
        pltpu.async_copy(bo.at[b], out_hbm.at[pl.ds(r0, _C)], souts[b])

    def wait_out(b):
        pltpu.make_async_copy(bo.at[b], out_hbm.at[pl.ds(0, _C)], souts[b]).wait()

    def compute(b):
        for jb in range(_VECS // _U):
            ts = [tab_v[pl.ds((jb * _U + u) * _LANES, _LANES)] for u in range(_U)]

            def rbody(r, c2):
                for u in range(_U):
                    off = (jb * _U + u) * _LANES
                    bo[b, r, pl.ds(off, _LANES)] = (
                        bi[b, r, pl.ds(off, _LANES)] + ts[u]
                    )
                return c2

            lax.fori_loop(0, _C, rbody, 0, unroll=2)

    start_in(0, 0)
    start_in(1, 1)
    for b in range(2):
        wait_in(b)
        compute(b)
        start_out(b, b)
        start_in(b + 2, b)

    def pair(p, carry):
        for b in range(2):
            g = p * 2 + b
            wait_in(b)
            wait_out(b)
            compute(b)
            start_out(g, b)
            nxt = jnp.minimum(g + 2, _NCH - 1)
            start_in(nxt, b)
        return carry

    lax.fori_loop(1, _NP, pair, 0)

    wait_in(0)
    wait_in(1)
    wait_out(0)
    wait_out(1)


def _sc_add(x, tab):
    mesh = plsc.VectorSubcoreMesh(core_axis_name="c", subcore_axis_name="s")
    return pl.kernel(
        _sc_body,
        out_type=jax.ShapeDtypeStruct((_SC_ROWS, _ROW), jnp.float32),
        mesh=mesh,
        compiler_params=pltpu.CompilerParams(use_tc_tiling_on_sc=True),
        scratch_types=[
            pltpu.VMEM((_ROW,), jnp.float32),
            pltpu.VMEM((2, _C, _ROW), jnp.float32),
            pltpu.VMEM((2, _C, _ROW), jnp.float32),
            pltpu.SemaphoreType.DMA,
            pltpu.SemaphoreType.DMA,
            pltpu.SemaphoreType.DMA,
            pltpu.SemaphoreType.DMA,
        ],
    )(x, tab)


def _tc_body(x_ref, t_ref, o_ref):
    o_ref[...] = x_ref[...] + t_ref[...]


def _tc_add(x, tab2):
    n = x.shape[0]
    return pl.pallas_call(
        _tc_body,
        grid=(n // _TC_BLOCK,),
        in_specs=[
            pl.BlockSpec((_TC_BLOCK, _ROW), lambda i: (i, 0)),
            pl.BlockSpec((1, _ROW), lambda i: (0, 0)),
        ],
        out_specs=pl.BlockSpec((_TC_BLOCK, _ROW), lambda i: (i, 0)),
        out_shape=jax.ShapeDtypeStruct((n, _ROW), jnp.float32),
    )(x, tab2)


def kernel(inputs, column_table):
    x = inputs.reshape(_BATCH, _ROW)
    tab = column_table.reshape(_ROW)
    out = _sc_add(x, tab)
    return out.reshape(_BATCH, _NUM_FEATURES, _EMBED_DIM)


# speedup vs baseline: 4.2042x; 3.2727x over previous
import jax
import jax.numpy as jnp
from jax import lax
from jax.experimental import pallas as pl
from jax.experimental.pallas import tpu as pltpu
from jax.experimental.pallas import tpu_sc as plsc

_F = 100
_D = 32
_BATCH = 16384
_ROW = _F * _D
_LANES = 16

_NC = 2
_NS = 16
_NW = _NC * _NS
_BPW = _BATCH // _NW

_FC = 2
_BC = 256
_NBC = _BPW // _BC
_NCH = (_F // _FC) * _NBC
_NP = _NCH // 2
_KB = _BC // _LANES


def _sc_body(x_hbm, tab_hbm, out_hbm, tab_v, bi, bo, sin0, sin1, so0, so1):
    sins = (sin0, sin1)
    souts = (so0, so1)
    wid = lax.axis_index("s") * _NC + lax.axis_index("c")
    col0 = wid * _BPW
    pltpu.sync_copy(tab_hbm, tab_v)

    def offs(g):
        f0 = (g // _NBC) * _FC
        b0 = col0 + (g % _NBC) * _BC
        return f0, b0

    def start_in(g, b):
        f0, b0 = offs(g)
        pltpu.async_copy(
            x_hbm.at[pl.ds(f0, _FC), :, pl.ds(b0, _BC)], bi.at[b], sins[b]
        )

    def wait_in(b):
        pltpu.make_async_copy(
            x_hbm.at[pl.ds(0, _FC), :, pl.ds(0, _BC)], bi.at[b], sins[b]
        ).wait()

    def start_out(g, b):
        f0, b0 = offs(g)
        pltpu.async_copy(
            bo.at[b], out_hbm.at[pl.ds(f0, _FC), :, pl.ds(b0, _BC)], souts[b]
        )

    def wait_out(b):
        pltpu.make_async_copy(
            bo.at[b], out_hbm.at[pl.ds(0, _FC), :, pl.ds(0, _BC)], souts[b]
        ).wait()

    def compute(g, b):
        f0, _ = offs(g)
        for f in range(_FC):
            def dbody(d, c2):
                t = tab_v[0, pl.ds(((f0 + f) * _D + d) * _LANES, _LANES)]
                for k in range(_KB):
                    sl = pl.ds(k * _LANES, _LANES)
                    bo[b, f, d, sl] = bi[b, f, d, sl] + t
                return c2

            lax.fori_loop(0, _D, dbody, 0, unroll=2)

    start_in(0, 0)
    start_in(1, 1)
    for b in range(2):
        wait_in(b)
        compute(b, b)
        start_out(b, b)
        start_in(b + 2, b)

    def pair(p, carry):
        for b in range(2):
            g = p * 2 + b
            wait_in(b)
            wait_out(b)
            compute(g, b)
            start_out(g, b)
            nxt = jnp.minimum(g + 2, _NCH - 1)
            start_in(nxt, b)
        return carry

    lax.fori_loop(1, _NP, pair, 0)

    wait_in(0)
    wait_in(1)
    wait_out(0)
    wait_out(1)


def kernel(inputs, column_table):
    xt = jnp.transpose(inputs, (1, 2, 0))
    tab = jnp.repeat(column_table.reshape(-1), _LANES).reshape(1, _ROW * _LANES)
    mesh = plsc.VectorSubcoreMesh(core_axis_name="c", subcore_axis_name="s")
    out_t = pl.kernel(
        _sc_body,
        out_type=jax.ShapeDtypeStruct((_F, _D, _BATCH), jnp.float32),
        mesh=mesh,
        scratch_types=[
            pltpu.VMEM((1, _ROW * _LANES), jnp.float32),
            pltpu.VMEM((2, _FC, _D, _BC), jnp.float32),
            pltpu.VMEM((2, _FC, _D, _BC), jnp.float32),
            pltpu.SemaphoreType.DMA,
            pltpu.SemaphoreType.DMA,
            pltpu.SemaphoreType.DMA,
            pltpu.SemaphoreType.DMA,
        ],
    )(xt, tab)
    return jnp.transpose(out_t, (2, 0, 1))
